# baseline (device time: 20669 ns/iter reference)
import jax
import jax.numpy as jnp
from jax import lax
from jax.experimental import pallas as pl
from jax.experimental.pallas import tpu as pltpu

N_DEV = 16
R = 4
BLK = 64
BUN = R * BLK


def kernel(x, w_mat):
    k_dim, m_per = x.shape
    n = w_mat.shape[1]

    def body(x_ref, w_ref, out_ref, comm1_ref, sbuf_ref, comm2_ref,
             ready_sems, send1_sems, recv1_sems, send2_sems, recv2_sems):
        my = lax.axis_index("i")
        a = my // R
        b = lax.rem(my, R)

        barrier_sem = pltpu.get_barrier_semaphore()
        pl.semaphore_signal(barrier_sem, inc=1)
        pl.semaphore_wait(barrier_sem, 1)

        for da in range(1, R):
            peer = lax.rem(a + da, R) * R + b
            pl.semaphore_signal(
                ready_sems.at[my], inc=1,
                device_id=(peer,), device_id_type=pl.DeviceIdType.MESH,
            )
        for db in range(1, R):
            peer = a * R + lax.rem(b + db, R)
            pl.semaphore_signal(
                ready_sems.at[my], inc=1,
                device_id=(peer,), device_id_type=pl.DeviceIdType.MESH,
            )

        rdmas1 = []
        for da in range(1, R):
            a_t = lax.rem(a + da, R)
            peer = a_t * R + b
            pl.semaphore_wait(ready_sems.at[peer], 1)
            rdma = pltpu.make_async_remote_copy(
                src_ref=x_ref.at[pl.ds(a_t * BUN, BUN), :],
                dst_ref=comm1_ref.at[a],
                send_sem=send1_sems.at[da],
                recv_sem=recv1_sems.at[da],
                device_id=(peer,),
                device_id_type=pl.DeviceIdType.MESH,
            )
            rdma.start()
            rdmas1.append(rdma)
        comm1_ref[a] = x_ref[pl.ds(a * BUN, BUN), :]

        for rdma in rdmas1:
            rdma.wait_recv()

        for a_src in range(R):
            for b_dst in range(R):
                sbuf_ref[b_dst, pl.ds(a_src * BLK, BLK), :] = (
                    comm1_ref[a_src, pl.ds(b_dst * BLK, BLK), :]
                )

        rdmas2 = []
        for db in range(1, R):
            b_t = lax.rem(b + db, R)
            peer = a * R + b_t
            pl.semaphore_wait(ready_sems.at[peer], 1)
            rdma = pltpu.make_async_remote_copy(
                src_ref=sbuf_ref.at[b_t],
                dst_ref=comm2_ref.at[b],
                send_sem=send2_sems.at[db],
                recv_sem=recv2_sems.at[db],
                device_id=(peer,),
                device_id_type=pl.DeviceIdType.MESH,
            )
            rdma.start()
            rdmas2.append(rdma)
        comm2_ref[b] = sbuf_ref[b]

        for rdma in rdmas2:
            rdma.wait_recv()

        x_rows = jnp.concatenate(
            [comm2_ref[s % R][(s // R) * BLK:(s // R) * BLK + BLK, :]
             for s in range(N_DEV)],
            axis=1,
        )
        y = jnp.dot(x_rows, w_ref[:, :], preferred_element_type=jnp.float32)
        out_ref[:, :] = y * jax.nn.sigmoid(y)

        for rdma in rdmas1:
            rdma.wait_send()
        for rdma in rdmas2:
            rdma.wait_send()

    return pl.pallas_call(
        body,
        out_shape=jax.ShapeDtypeStruct((BLK, n), jnp.float32),
        in_specs=[
            pl.BlockSpec(memory_space=pltpu.VMEM),
            pl.BlockSpec(memory_space=pltpu.VMEM),
        ],
        out_specs=pl.BlockSpec(memory_space=pltpu.VMEM),
        scratch_shapes=[
            pltpu.VMEM((R, BUN, m_per), jnp.float32),
            pltpu.VMEM((R, BUN, m_per), jnp.float32),
            pltpu.VMEM((R, BUN, m_per), jnp.float32),
            pltpu.SemaphoreType.REGULAR((N_DEV,)),
            pltpu.SemaphoreType.DMA((R,)),
            pltpu.SemaphoreType.DMA((R,)),
            pltpu.SemaphoreType.DMA((R,)),
            pltpu.SemaphoreType.DMA((R,)),
        ],
        compiler_params=pltpu.CompilerParams(collective_id=0),
    )(x, w_mat)


# device time: 15903 ns/iter; 1.2997x vs baseline; 1.2997x over previous
import jax
import jax.numpy as jnp
from jax import lax
from jax.experimental import pallas as pl
from jax.experimental.pallas import tpu as pltpu

N_DEV = 16
BLK = 64
GRP = 4
CHUNK = GRP * BLK


def kernel(x, w_mat):
    k_dim, m_per = x.shape
    n = w_mat.shape[1]

    def body(x_ref, w_ref, out_ref, comm_ref, send_sems, recv_sems):
        my = lax.axis_index("i")

        barrier_sem = pltpu.get_barrier_semaphore()
        for d in range(1, N_DEV):
            peer = lax.rem(my + d, N_DEV)
            pl.semaphore_signal(
                barrier_sem, inc=1,
                device_id=(peer,), device_id_type=pl.DeviceIdType.MESH,
            )
        pl.semaphore_wait(barrier_sem, N_DEV - 1)

        diag = pltpu.make_async_copy(
            x_ref.at[pl.ds(my * BLK, BLK), :],
            comm_ref.at[pl.ds(my * BLK, BLK), :],
            recv_sems.at[my],
        )
        diag.start()

        rdmas = []
        for d in range(1, N_DEV):
            peer = lax.rem(my + d, N_DEV)
            rdma = pltpu.make_async_remote_copy(
                src_ref=x_ref.at[pl.ds(peer * BLK, BLK), :],
                dst_ref=comm_ref.at[pl.ds(my * BLK, BLK), :],
                send_sem=send_sems.at[d],
                recv_sem=recv_sems.at[my],
                device_id=(peer,),
                device_id_type=pl.DeviceIdType.MESH,
            )
            rdma.start()
            rdmas.append(rdma)

        y = jnp.zeros((BLK, n), dtype=jnp.float32)
        for g in range(N_DEV // GRP):
            for s in range(g * GRP, (g + 1) * GRP):
                pltpu.make_async_copy(
                    comm_ref.at[pl.ds(s * BLK, BLK), :],
                    comm_ref.at[pl.ds(s * BLK, BLK), :],
                    recv_sems.at[s],
                ).wait()
            xg = jnp.concatenate(
                [comm_ref[s * BLK:(s + 1) * BLK, :]
                 for s in range(g * GRP, (g + 1) * GRP)],
                axis=1,
            )
            y = y + jnp.dot(
                xg, w_ref[g * CHUNK:(g + 1) * CHUNK, :],
                preferred_element_type=jnp.float32,
            )

        out_ref[:, :] = y * jax.nn.sigmoid(y)

        for rdma in rdmas:
            rdma.wait_send()

    return pl.pallas_call(
        body,
        out_shape=jax.ShapeDtypeStruct((BLK, n), jnp.float32),
        in_specs=[
            pl.BlockSpec(memory_space=pltpu.VMEM),
            pl.BlockSpec(memory_space=pltpu.VMEM),
        ],
        out_specs=pl.BlockSpec(memory_space=pltpu.VMEM),
        scratch_shapes=[
            pltpu.VMEM((k_dim, m_per), jnp.float32),
            pltpu.SemaphoreType.DMA((N_DEV,)),
            pltpu.SemaphoreType.DMA((N_DEV,)),
        ],
        compiler_params=pltpu.CompilerParams(collective_id=0),
    )(x, w_mat)
